# baseline (speedup 1.0000x reference)
import functools

import jax
import jax.numpy as jnp
from jax import lax
from jax.experimental import pallas as pl
from jax.experimental.pallas import tpu as pltpu
from jax.experimental.pallas import tpu_sc as plsc

DICT_SIZE = 1000000
SEQ_LENGTH = 200
EMB_DIM = 32
INTERMEDIATE_DIM = 1024
BATCH = 4096
BASE_FREQ = 10000.0
FLAT_DIM = SEQ_LENGTH * EMB_DIM

TOTAL_ROWS = BATCH * SEQ_LENGTH
NUM_WORKERS = 32
IDX_ROW = 128
GATHERS_PER_CHUNK = 8
CHUNK = IDX_ROW * GATHERS_PER_CHUNK
ROWS_PER_WORKER = TOTAL_ROWS // NUM_WORKERS
CHUNKS_PER_WORKER = ROWS_PER_WORKER // CHUNK
IDX_ROWS_PER_WORKER = ROWS_PER_WORKER // IDX_ROW

BM = 256
TB = 256


def _idx_t_body(x_ref, out_ref):
    out_ref[...] = x_ref[...].T


def _idx_transpose(idx_T):
    return pl.pallas_call(
        _idx_t_body,
        grid=(BATCH // TB,),
        in_specs=[pl.BlockSpec((SEQ_LENGTH, TB), lambda i: (0, i))],
        out_specs=pl.BlockSpec((TB, SEQ_LENGTH), lambda i: (i, 0)),
        out_shape=jax.ShapeDtypeStruct((BATCH, SEQ_LENGTH), jnp.int32),
    )(idx_T)


TTB = 16384


def _tab_t_body(x_ref, out_ref):
    out_ref[...] = x_ref[...].T


def _tab_transpose(table_T):
    return pl.pallas_call(
        _tab_t_body,
        grid=(pl.cdiv(DICT_SIZE, TTB),),
        in_specs=[pl.BlockSpec((EMB_DIM, TTB), lambda i: (0, i))],
        out_specs=pl.BlockSpec((TTB, EMB_DIM), lambda i: (i, 0)),
        out_shape=jax.ShapeDtypeStruct((DICT_SIZE, EMB_DIM), jnp.float32),
    )(table_T)


def _sc_gather_body(idx_hbm, table_hbm, out_hbm, idx_v, rows_v, sem):
    c = lax.axis_index("c")
    s = lax.axis_index("s")
    wid = s * 2 + c
    idx_row_base = wid * IDX_ROWS_PER_WORKER

    def chunk_body(i, carry):
        row0 = idx_row_base + i * GATHERS_PER_CHUNK
        pltpu.sync_copy(idx_hbm.at[pl.ds(row0, GATHERS_PER_CHUNK)], idx_v)
        copies = []
        for j in range(GATHERS_PER_CHUNK):
            copies.append(
                pltpu.async_copy(
                    table_hbm.at[idx_v.at[j]],
                    rows_v.at[pl.ds(j * IDX_ROW, IDX_ROW)],
                    sem,
                )
            )
        for cp in copies:
            cp.wait()
        pltpu.sync_copy(rows_v, out_hbm.at[pl.ds(row0 * IDX_ROW, CHUNK)])
        return carry

    lax.fori_loop(0, CHUNKS_PER_WORKER, chunk_body, 0)


@jax.jit
def _sc_gather(idx2d, table):
    mesh = plsc.VectorSubcoreMesh(core_axis_name="c", subcore_axis_name="s")
    return pl.kernel(
        _sc_gather_body,
        out_type=jax.ShapeDtypeStruct((TOTAL_ROWS, EMB_DIM), jnp.float32),
        mesh=mesh,
        scratch_types=[
            pltpu.VMEM((GATHERS_PER_CHUNK, IDX_ROW), jnp.int32),
            pltpu.VMEM((CHUNK, EMB_DIM), jnp.float32),
            pltpu.SemaphoreType.DMA,
        ],
        compiler_params=pltpu.CompilerParams(use_tc_tiling_on_sc=False),
    )(idx2d, table)


def _mlp_body(x_ref, pe_ref, w1_ref, b1_ref, w2_ref, b2_ref, out_ref):
    x = x_ref[...] + pe_ref[...]
    h = jnp.dot(x, w1_ref[...], preferred_element_type=jnp.float32)
    h = jnp.maximum(h + b1_ref[...], 0.0)
    h = jnp.dot(h, w2_ref[...], preferred_element_type=jnp.float32)
    h = jnp.maximum(h + b2_ref[...], 0.0)
    m = jnp.max(h, axis=-1, keepdims=True)
    e = jnp.exp(h - m)
    lse = jnp.log(jnp.sum(e, axis=-1, keepdims=True)) + m
    out_ref[...] = h - lse


def _mlp(x, pe_flat, W1, b1, W2, b2):
    grid = (BATCH // BM,)
    return pl.pallas_call(
        _mlp_body,
        grid=grid,
        in_specs=[
            pl.BlockSpec((BM, FLAT_DIM), lambda i: (i, 0)),
            pl.BlockSpec((1, FLAT_DIM), lambda i: (0, 0)),
            pl.BlockSpec((FLAT_DIM, INTERMEDIATE_DIM), lambda i: (0, 0)),
            pl.BlockSpec((1, INTERMEDIATE_DIM), lambda i: (0, 0)),
            pl.BlockSpec((INTERMEDIATE_DIM, INTERMEDIATE_DIM), lambda i: (0, 0)),
            pl.BlockSpec((1, INTERMEDIATE_DIM), lambda i: (0, 0)),
        ],
        out_specs=pl.BlockSpec((BM, INTERMEDIATE_DIM), lambda i: (i, 0)),
        out_shape=jax.ShapeDtypeStruct((BATCH, INTERMEDIATE_DIM), jnp.float32),
    )(x, pe_flat, W1, b1, W2, b2)


def _positional_encoding_flat():
    pos = jnp.arange(SEQ_LENGTH, dtype=jnp.float32)[:, None]
    i = jnp.arange(0, EMB_DIM, 2, dtype=jnp.float32)[None, :]
    angle = pos / jnp.power(BASE_FREQ, i / EMB_DIM)
    pe = jnp.zeros((SEQ_LENGTH, EMB_DIM), dtype=jnp.float32)
    pe = pe.at[:, 0::2].set(jnp.sin(angle))
    pe = pe.at[:, 1::2].set(jnp.cos(angle))
    return pe.reshape(1, FLAT_DIM)


def kernel(indexed_sentences, emb_table, W1, b1, W2, b2):
    idx_T = indexed_sentences.astype(jnp.int32).T
    idx_bt = _idx_transpose(idx_T)
    idx2d = idx_bt.reshape(TOTAL_ROWS // IDX_ROW, IDX_ROW)
    table_rm = _tab_transpose(emb_table.T)
    emb_rows = _sc_gather(idx2d, table_rm)
    x = emb_rows.reshape(BATCH, FLAT_DIM)
    pe_flat = _positional_encoding_flat()
    return _mlp(
        x, pe_flat, W1, b1.reshape(1, -1), W2, b2.reshape(1, -1)
    )


# speedup vs baseline: 1.2937x; 1.2937x over previous
import functools

import jax
import jax.numpy as jnp
from jax import lax
from jax.experimental import pallas as pl
from jax.experimental.pallas import tpu as pltpu
from jax.experimental.pallas import tpu_sc as plsc

DICT_SIZE = 1000000
SEQ_LENGTH = 200
EMB_DIM = 32
INTERMEDIATE_DIM = 1024
BATCH = 4096
BASE_FREQ = 10000.0
FLAT_DIM = SEQ_LENGTH * EMB_DIM

TOTAL_ROWS = BATCH * SEQ_LENGTH
NUM_WORKERS = 32
IDX_ROW = 128
GATHERS_PER_CHUNK = 8
CHUNK = IDX_ROW * GATHERS_PER_CHUNK
ROWS_PER_WORKER = TOTAL_ROWS // NUM_WORKERS
CHUNKS_PER_WORKER = ROWS_PER_WORKER // CHUNK
IDX_ROWS_PER_WORKER = ROWS_PER_WORKER // IDX_ROW

BM = 256
TB = 256


def _idx_t_body(x_ref, out_ref):
    out_ref[...] = x_ref[...].T


def _idx_transpose(idx_T):
    return pl.pallas_call(
        _idx_t_body,
        grid=(BATCH // TB,),
        in_specs=[pl.BlockSpec((SEQ_LENGTH, TB), lambda i: (0, i))],
        out_specs=pl.BlockSpec((TB, SEQ_LENGTH), lambda i: (i, 0)),
        out_shape=jax.ShapeDtypeStruct((BATCH, SEQ_LENGTH), jnp.int32),
    )(idx_T)


TTB = 16384


def _tab_t_body(x_ref, out_ref):
    t4 = x_ref[...].T.reshape(TTB // 4, 4, EMB_DIM)
    for j in range(4):
        out_ref[:, 32 * j:32 * (j + 1)] = t4[:, j, :]


def _tab_transpose(table_T):
    return pl.pallas_call(
        _tab_t_body,
        grid=(pl.cdiv(DICT_SIZE, TTB),),
        in_specs=[pl.BlockSpec((EMB_DIM, TTB), lambda i: (0, i))],
        out_specs=pl.BlockSpec((TTB // 4, 128), lambda i: (i, 0)),
        out_shape=jax.ShapeDtypeStruct((DICT_SIZE // 4, 128), jnp.float32),
    )(table_T)


def _sc_gather_body(idx_hbm, table_hbm, out_hbm, idx_v, rows_v, sem):
    c = lax.axis_index("c")
    s = lax.axis_index("s")
    wid = s * 2 + c
    idx_row_base = wid * IDX_ROWS_PER_WORKER

    def chunk_body(i, carry):
        row0 = idx_row_base + i * GATHERS_PER_CHUNK
        pltpu.sync_copy(idx_hbm.at[pl.ds(row0, GATHERS_PER_CHUNK)], idx_v)
        copies = []
        for j in range(GATHERS_PER_CHUNK):
            copies.append(
                pltpu.async_copy(
                    table_hbm.at[idx_v.at[j]],
                    rows_v.at[pl.ds(j * IDX_ROW, IDX_ROW)],
                    sem,
                )
            )
        for cp in copies:
            cp.wait()
        pltpu.sync_copy(rows_v, out_hbm.at[pl.ds(row0 * IDX_ROW, CHUNK)])
        return carry

    lax.fori_loop(0, CHUNKS_PER_WORKER, chunk_body, 0)


@jax.jit
def _sc_gather(idx2d, table):
    mesh = plsc.VectorSubcoreMesh(core_axis_name="c", subcore_axis_name="s")
    return pl.kernel(
        _sc_gather_body,
        out_type=jax.ShapeDtypeStruct((TOTAL_ROWS, EMB_DIM), jnp.float32),
        mesh=mesh,
        scratch_types=[
            pltpu.VMEM((GATHERS_PER_CHUNK, IDX_ROW), jnp.int32),
            pltpu.VMEM((CHUNK, EMB_DIM), jnp.float32),
            pltpu.SemaphoreType.DMA,
        ],
        compiler_params=pltpu.CompilerParams(use_tc_tiling_on_sc=False),
    )(idx2d, table)


def _mlp_body(x_ref, pe_ref, w1_ref, b1_ref, w2_ref, b2_ref, out_ref):
    x = x_ref[...] + pe_ref[...]
    h = jnp.dot(x, w1_ref[...], preferred_element_type=jnp.float32)
    h = jnp.maximum(h + b1_ref[...], 0.0)
    h = jnp.dot(h, w2_ref[...], preferred_element_type=jnp.float32)
    h = jnp.maximum(h + b2_ref[...], 0.0)
    m = jnp.max(h, axis=-1, keepdims=True)
    e = jnp.exp(h - m)
    lse = jnp.log(jnp.sum(e, axis=-1, keepdims=True)) + m
    out_ref[...] = h - lse


def _mlp(x, pe_flat, W1, b1, W2, b2):
    grid = (BATCH // BM,)
    return pl.pallas_call(
        _mlp_body,
        grid=grid,
        in_specs=[
            pl.BlockSpec((BM, FLAT_DIM), lambda i: (i, 0)),
            pl.BlockSpec((1, FLAT_DIM), lambda i: (0, 0)),
            pl.BlockSpec((FLAT_DIM, INTERMEDIATE_DIM), lambda i: (0, 0)),
            pl.BlockSpec((1, INTERMEDIATE_DIM), lambda i: (0, 0)),
            pl.BlockSpec((INTERMEDIATE_DIM, INTERMEDIATE_DIM), lambda i: (0, 0)),
            pl.BlockSpec((1, INTERMEDIATE_DIM), lambda i: (0, 0)),
        ],
        out_specs=pl.BlockSpec((BM, INTERMEDIATE_DIM), lambda i: (i, 0)),
        out_shape=jax.ShapeDtypeStruct((BATCH, INTERMEDIATE_DIM), jnp.float32),
    )(x, pe_flat, W1, b1, W2, b2)


def _positional_encoding_flat():
    pos = jnp.arange(SEQ_LENGTH, dtype=jnp.float32)[:, None]
    i = jnp.arange(0, EMB_DIM, 2, dtype=jnp.float32)[None, :]
    angle = pos / jnp.power(BASE_FREQ, i / EMB_DIM)
    pe = jnp.zeros((SEQ_LENGTH, EMB_DIM), dtype=jnp.float32)
    pe = pe.at[:, 0::2].set(jnp.sin(angle))
    pe = pe.at[:, 1::2].set(jnp.cos(angle))
    return pe.reshape(1, FLAT_DIM)


def kernel(indexed_sentences, emb_table, W1, b1, W2, b2):
    idx_T = indexed_sentences.astype(jnp.int32).T
    idx_bt = _idx_transpose(idx_T)
    idx2d = idx_bt.reshape(TOTAL_ROWS // IDX_ROW, IDX_ROW)
    table_rm = _tab_transpose(emb_table.T).reshape(DICT_SIZE, EMB_DIM)
    emb_rows = _sc_gather(idx2d, table_rm)
    x = emb_rows.reshape(BATCH, FLAT_DIM)
    pe_flat = _positional_encoding_flat()
    return _mlp(
        x, pe_flat, W1, b1.reshape(1, -1), W2, b2.reshape(1, -1)
    )
